# Initial kernel scaffold; baseline (speedup 1.0000x reference)
import jax
import jax.numpy as jnp
from jax.experimental import pallas as pl


def kernel(feats, labels, dist_mat, max_triplet):
    raise NotImplementedError("write your pallas kernel here")


# speedup vs baseline: 85.9954x; 85.9954x over previous
import functools

import jax
import jax.numpy as jnp
from jax import lax
from jax.experimental import pallas as pl
from jax.experimental.pallas import tpu as pltpu
from jax.experimental.pallas import tpu_sc as plsc

NCLS = 28
NCP = 32
M = 200
MPAD = 208
GPAD = 224
NW = 32
N = 2 * 16 * 128 * 128
CHUNK = N // NW
VREGS = CHUNK // 16
SENT = 0x3FFFFFFF

_mesh = functools.partial(
    plsc.VectorSubcoreMesh, core_axis_name="c", subcore_axis_name="s")
_cp = pltpu.CompilerParams(needs_layout_passes=False)
_cp_untiled = pltpu.CompilerParams(needs_layout_passes=False,
                                   use_tc_tiling_on_sc=False)


def _wid():
    return lax.axis_index("s") * 2 + lax.axis_index("c")


def _lane():
    return lax.iota(jnp.int32, 16)


def _k1_body(labels_hbm, loc_cnt_hbm, loc_idx_hbm, lab_v, buf_v, cnt_v, sem):
    wid = _wid()
    base = wid * CHUNK
    pltpu.sync_copy(labels_hbm.at[pl.ds(base, CHUNK)], lab_v)
    z16 = jnp.zeros((16,), jnp.int32)
    cnt_v[pl.ds(0, 16)] = z16
    cnt_v[pl.ds(16, 16)] = z16
    lane = _lane()

    def step(i, _):
        v = lab_v[pl.ds(i * 16, 16)]
        occ, lastm = plsc.scan_count(v)
        basec = plsc.load_gather(cnt_v, [v])
        pos = basec + occ - 1
        keep = pos < M
        gidx = base + i * 16 + lane
        plsc.store_scatter(buf_v, [v, jnp.minimum(pos, M - 1)], gidx,
                           mask=keep)
        plsc.store_scatter(cnt_v, [v], basec + occ, mask=lastm)
        return 0

    lax.fori_loop(0, VREGS, step, 0)
    pltpu.sync_copy(cnt_v, loc_cnt_hbm.at[wid])
    copies = [pltpu.async_copy(buf_v.at[c], loc_idx_hbm.at[c, wid], sem)
              for c in range(NCLS)]
    for d in copies:
        d.wait()


def _k1(labels_flat):
    kern = pl.kernel(
        _k1_body,
        out_type=(jax.ShapeDtypeStruct((NW, NCP), jnp.int32),
                  jax.ShapeDtypeStruct((NCLS, NW, M), jnp.int32)),
        mesh=_mesh(),
        compiler_params=_cp,
        scratch_types=[pltpu.VMEM((CHUNK,), jnp.int32),
                       pltpu.VMEM((NCLS, M), jnp.int32),
                       pltpu.VMEM((NCP,), jnp.int32),
                       pltpu.SemaphoreType.DMA],
    )
    return kern(labels_flat)


def _k15_body(loc_cnt_hbm, loc_idx_hbm, glob_hbm, cnt_hbm,
              lc_v, cls_v, pre_v, glob_v, cw_v, sem):
    wid = _wid()
    lane = _lane()
    z16 = jnp.zeros((16,), jnp.int32)
    pltpu.sync_copy(loc_cnt_hbm, lc_v)

    @pl.when(wid == 0)
    def _():
        acc0 = z16
        acc1 = z16
        for w in range(NW):
            acc0 = acc0 + lc_v[w, pl.ds(0, 16)]
            acc1 = acc1 + lc_v[w, pl.ds(16, 16)]
        cw_v[pl.ds(0, 16)] = acc0
        cw_v[pl.ds(16, 16)] = acc1
        pltpu.sync_copy(cw_v, cnt_hbm)

    @pl.when(wid < NCLS)
    def _():
        c = wid
        pltpu.sync_copy(loc_idx_hbm.at[c], cls_v)
        fc = jnp.full((16,), c, jnp.int32)
        cw0 = plsc.load_gather(lc_v, [lane, fc])
        cw1 = plsc.load_gather(lc_v, [lane + 16, fc])
        t0 = jnp.sum(cw0)
        incl0 = plsc.cumsum(cw0)
        incl1 = plsc.cumsum(cw1) + t0
        total = t0 + jnp.sum(cw1)
        pre_v[pl.ds(0, 16)] = incl0 - cw0
        pre_v[pl.ds(16, 16)] = incl1 - cw1
        cap = jnp.minimum(total, M)

        e0 = incl0 - cw0
        e1 = incl1 - cw1

        def grp(g, _):
            j = g * 16 + lane
            acc = jnp.zeros((16,), jnp.int32)
            for w in range(16):
                acc = acc + (j >= e0[w]).astype(jnp.int32)
                acc = acc + (j >= e1[w]).astype(jnp.int32)
            wsel = acc - 1
            local = j - plsc.load_gather(pre_v, [wsel])
            local = jnp.clip(local, 0, M - 1)
            val = plsc.load_gather(cls_v, [wsel, local])
            val = jnp.where(j < cap, val, SENT)
            glob_v[pl.ds(g * 16, 16)] = val
            return 0

        lax.fori_loop(0, MPAD // 16, grp, 0)
        pltpu.sync_copy(glob_v, glob_hbm.at[c])


def _k15(loc_cnt, loc_idx):
    kern = pl.kernel(
        _k15_body,
        out_type=(jax.ShapeDtypeStruct((NCLS, MPAD), jnp.int32),
                  jax.ShapeDtypeStruct((NCP,), jnp.int32)),
        mesh=_mesh(),
        compiler_params=_cp,
        scratch_types=[pltpu.VMEM((NW, NCP), jnp.int32),
                       pltpu.VMEM((NW, M), jnp.int32),
                       pltpu.VMEM((NCP,), jnp.int32),
                       pltpu.VMEM((MPAD,), jnp.int32),
                       pltpu.VMEM((NCP,), jnp.int32),
                       pltpu.SemaphoreType.DMA],
    )
    return kern(loc_cnt, loc_idx)


def _k2_body(glob_hbm, cnt_hbm, dm_hbm, featsf_hbm, mt_hbm, res_hbm,
             glob_v, cnt_v, dm_v, mt_v,
             oidx_a, oidx_p, oidx_n, ocls_p, ocls_n,
             rows_a, rows_p, rows_n, res_v, sem):
    wid = _wid()
    lane = _lane()
    z16 = jnp.zeros((16,), jnp.int32)
    zf16 = jnp.zeros((16,), jnp.float32)

    @pl.when(wid >= NCLS - 1)
    def _():
        res_v[...] = zf16
        pltpu.sync_copy(res_v, res_hbm.at[wid])

    @pl.when(wid < NCLS - 1)
    def _():
        ii = wid + 1
        pltpu.sync_copy(glob_hbm, glob_v)
        pltpu.sync_copy(cnt_hbm, cnt_v)
        pltpu.sync_copy(dm_hbm, dm_v)
        pltpu.sync_copy(mt_hbm, mt_v)
        iiv = jnp.full((16,), ii, jnp.int32)
        c0 = lane
        c1 = lane + 16
        c1c = jnp.minimum(c1, NCLS - 1)
        cnt0 = cnt_v[pl.ds(0, 16)]
        cnt1 = cnt_v[pl.ds(16, 16)]
        dr0 = plsc.load_gather(dm_v, [iiv, c0])
        dr1 = plsc.load_gather(dm_v, [iiv, c1])
        adv0 = jnp.where(dr0 == 0.0, jnp.float32(256.0), dr0)
        adv1 = jnp.where(dr1 == 0.0, jnp.float32(256.0), dr1)
        exist0 = jnp.logical_and(cnt0 > 0, c0 != 0)
        exist1 = jnp.logical_and(cnt1 > 0, c1 < NCLS)
        inf = jnp.float32(jnp.inf)
        mn = jnp.minimum(jnp.min(jnp.where(exist0, adv0, inf)),
                         jnp.min(jnp.where(exist1, adv1, inf)))
        sel0 = jnp.logical_and(exist0, adv0 == mn)
        sel1 = jnp.logical_and(exist1, adv1 == mn)
        neg0 = (c0 != 0) & (c0 != ii) & jnp.logical_not(sel0)
        neg1 = (c1 < NCLS) & (c1 != ii) & jnp.logical_not(sel1)
        pos_tot = (jnp.sum(jnp.where(sel0, cnt0, 0))
                   + jnp.sum(jnp.where(sel1, cnt1, 0)))
        neg_tot = (jnp.sum(jnp.where(neg0, cnt0, 0))
                   + jnp.sum(jnp.where(neg1, cnt1, 0)))
        cnt_ii = jnp.sum(jnp.where(c0 == ii, cnt0, 0)) + \
            jnp.sum(jnp.where(c1 == ii, cnt1, 0))
        mt = jnp.min(mt_v[...])
        ms = jnp.minimum(jnp.minimum(cnt_ii, pos_tot),
                         jnp.minimum(neg_tot, mt))

        for j2 in range(2):
            for g in range(7):
                oidx_a[j2, pl.ds(g * 16, 16)] = z16
                oidx_p[j2, pl.ds(g * 16, 16)] = z16
                oidx_n[j2, pl.ds(g * 16, 16)] = z16
        for g in range(MPAD // 16):
            ocls_p[pl.ds(g * 16, 16)] = z16
            ocls_n[pl.ds(g * 16, 16)] = z16

        def afill(g, _):
            kvec = g * 16 + lane
            val = plsc.load_gather(glob_v, [iiv, jnp.minimum(kvec, MPAD - 1)])
            mc = jnp.minimum(val, N - 1)
            row = (kvec >= 112).astype(jnp.int32)
            col = kvec - row * 112
            plsc.store_scatter(oidx_a, [row, col], mc)
            return 0

        lax.fori_loop(0, GPAD // 16, afill, 0)

        def run_merge(memb0, memb1, oidx_ref, ocls_ref):
            h0 = jnp.where(memb0,
                           plsc.load_gather(glob_v, [c0, z16]), SENT)
            h1 = jnp.where(memb1,
                           plsc.load_gather(glob_v, [c1c, z16]), SENT)
            lane0 = lane == 0

            def mstep(k, carry):
                h0, h1, p0, p1 = carry
                m = jnp.minimum(jnp.min(h0), jnp.min(h1))
                e0 = h0 == m
                e1 = h1 == m
                f0 = plsc.all_reduce_ffs(e0)
                f1 = plsc.all_reduce_ffs(e1)
                has0 = jnp.any(e0)
                cvec = jnp.where(has0, f0, f1 + 16)
                mc = jnp.full((16,), jnp.minimum(m, N - 1), jnp.int32)
                row = (k >= 112).astype(jnp.int32)
                col = k - row * 112
                plsc.store_scatter(
                    oidx_ref, [jnp.full((16,), row, jnp.int32),
                               jnp.full((16,), col, jnp.int32)],
                    mc, mask=lane0)
                plsc.store_scatter(ocls_ref, [jnp.full((16,), k, jnp.int32)],
                                   cvec, mask=lane0)
                a0 = jnp.logical_and(memb0, c0 == cvec)
                a1 = jnp.logical_and(memb1, c1 == cvec)
                p0n = p0 + a0.astype(jnp.int32)
                p1n = p1 + a1.astype(jnp.int32)
                g0 = plsc.load_gather(glob_v, [c0, jnp.minimum(p0n, MPAD - 1)])
                g1 = plsc.load_gather(glob_v,
                                      [c1c, jnp.minimum(p1n, MPAD - 1)])
                h0n = jnp.where(a0, g0, h0)
                h1n = jnp.where(a1, g1, h1)
                return h0n, h1n, p0n, p1n

            lax.fori_loop(0, M, mstep, (h0, h1, z16, z16))

        run_merge(sel0, sel1, oidx_p, ocls_p)
        run_merge(neg0, neg1, oidx_n, ocls_n)

        descs = []
        for (oix, rws) in ((oidx_a, rows_a), (oidx_p, rows_p),
                           (oidx_n, rows_n)):
            for j2 in range(2):
                descs.append(pltpu.async_copy(
                    featsf_hbm.at[oix.at[j2]], rws.at[j2], sem))
        for d in descs:
            d.wait()

        ms_v = jnp.full((16,), ms, jnp.int32)

        def grp(g, tl_sum):
            kvec = g * 16 + lane
            row = (kvec >= 112).astype(jnp.int32)
            col = kvec - row * 112
            accp = zf16
            accn = zf16
            for ch in range(32):
                fch = jnp.full((16,), ch, jnp.int32)
                va = plsc.load_gather(rows_a, [row, col, fch])
                vp = plsc.load_gather(rows_p, [row, col, fch])
                vn = plsc.load_gather(rows_n, [row, col, fch])
                accp = accp + va * vp
                accn = accn + va * vn
            ocp = ocls_p[pl.ds(g * 16, 16)]
            ocn = ocls_n[pl.ds(g * 16, 16)]
            dpos = plsc.load_gather(dm_v, [iiv, ocp])
            dneg = plsc.load_gather(dm_v, [iiv, ocn])
            d_ap = 1.0 - accp
            d_an = 1.0 - accn
            tl = jnp.maximum(
                d_ap - d_an + 0.1 + (dneg - dpos) * 0.125, 0.0)
            tl = jnp.where(kvec < ms_v, tl, 0.0)
            return tl_sum + jnp.sum(tl)

        tl_sum = lax.fori_loop(0, MPAD // 16, grp, jnp.float32(0.0))
        active = ms > 0
        msf_v = jnp.full((16,), jnp.maximum(ms, 1).astype(jnp.float32))
        tl_mean = jnp.where(jnp.full((16,), active),
                            jnp.full((16,), tl_sum) / msf_v, zf16)
        act_f = jnp.where(active, jnp.float32(1.0), jnp.float32(0.0))
        vals = (tl_mean * (lane == 0).astype(jnp.float32)
                + act_f * (lane == 1).astype(jnp.float32))
        res_v[...] = vals
        pltpu.sync_copy(res_v, res_hbm.at[wid])


def _k2(glob, cnt, dm_pad, featsf, mt_vec):
    kern = pl.kernel(
        _k2_body,
        out_type=jax.ShapeDtypeStruct((NW, 16), jnp.float32),
        mesh=_mesh(),
        compiler_params=_cp_untiled,
        scratch_types=[pltpu.VMEM((NCLS, MPAD), jnp.int32),
                       pltpu.VMEM((NCP,), jnp.int32),
                       pltpu.VMEM((NCP, NCP), jnp.float32),
                       pltpu.VMEM((16,), jnp.int32),
                       pltpu.VMEM((2, 112), jnp.int32),
                       pltpu.VMEM((2, 112), jnp.int32),
                       pltpu.VMEM((2, 112), jnp.int32),
                       pltpu.VMEM((MPAD,), jnp.int32),
                       pltpu.VMEM((MPAD,), jnp.int32),
                       pltpu.VMEM((2, 112, 32), jnp.float32),
                       pltpu.VMEM((2, 112, 32), jnp.float32),
                       pltpu.VMEM((2, 112, 32), jnp.float32),
                       pltpu.VMEM((16,), jnp.float32),
                       pltpu.SemaphoreType.DMA],
    )
    return kern(glob, cnt, dm_pad, featsf, mt_vec)


def _k3_body(res_hbm, out_hbm, res_v, out_v):
    wid = _wid()
    lane = _lane()

    @pl.when(wid == 0)
    def _():
        pltpu.sync_copy(res_hbm, res_v)
        z16 = jnp.zeros((16,), jnp.int32)
        o16 = jnp.full((16,), 1, jnp.int32)
        tl0 = plsc.load_gather(res_v, [lane, z16])
        tl1 = plsc.load_gather(res_v, [lane + 16, z16])
        ac0 = plsc.load_gather(res_v, [lane, o16])
        ac1 = plsc.load_gather(res_v, [lane + 16, o16])
        total = jnp.sum(tl0) + jnp.sum(tl1)
        count = jnp.sum(ac0) + jnp.sum(ac1)
        total_v = jnp.full((16,), total)
        count_v = jnp.full((16,), count)
        result_v = jnp.where(count_v == 0.0, jnp.float32(0.0),
                             total_v / jnp.maximum(count_v, 1.0))
        out_v[...] = result_v
        pltpu.sync_copy(out_v, out_hbm)


def _k3(res):
    kern = pl.kernel(
        _k3_body,
        out_type=jax.ShapeDtypeStruct((16,), jnp.float32),
        mesh=_mesh(),
        compiler_params=_cp,
        scratch_types=[pltpu.VMEM((NW, 16), jnp.float32),
                       pltpu.VMEM((16,), jnp.float32)],
    )
    return kern(res)


def kernel(feats, labels, dist_mat, max_triplet):
    lab = labels[:, ::2, ::2, ::2].astype(jnp.int32)
    labels_flat = lab.reshape(-1)
    featsf = jnp.transpose(feats, (0, 2, 3, 4, 1)).reshape(-1, feats.shape[1])
    dm_pad = jnp.zeros((NCP, NCP), jnp.float32).at[:NCLS, :NCLS].set(
        dist_mat.astype(jnp.float32))
    mt_vec = jnp.full((16,), max_triplet, jnp.int32)

    loc_cnt, loc_idx = _k1(labels_flat)
    glob, cnt = _k15(loc_cnt, loc_idx)
    res = _k2(glob, cnt, dm_pad, featsf, mt_vec)
    out = _k3(res)
    return out[0]


# speedup vs baseline: 87.8629x; 1.0217x over previous
import functools

import jax
import jax.numpy as jnp
from jax import lax
from jax.experimental import pallas as pl
from jax.experimental.pallas import tpu as pltpu
from jax.experimental.pallas import tpu_sc as plsc

NCLS = 28
NCP = 32
M = 200
MPAD = 208
GPAD = 224
NW = 32
N = 2 * 16 * 128 * 128
CHUNK = N // NW
VREGS = CHUNK // 16
SENT = 0x3FFFFFFF

_mesh = functools.partial(
    plsc.VectorSubcoreMesh, core_axis_name="c", subcore_axis_name="s")
_cp = pltpu.CompilerParams(needs_layout_passes=False)
_cp_untiled = pltpu.CompilerParams(needs_layout_passes=False,
                                   use_tc_tiling_on_sc=False)


def _wid():
    return lax.axis_index("s") * 2 + lax.axis_index("c")


def _lane():
    return lax.iota(jnp.int32, 16)


def _k1_body(labels_hbm, loc_cnt_hbm, loc_idx_hbm, lab_v, buf_v, cnt_v, sem):
    wid = _wid()
    base = wid * CHUNK
    pltpu.sync_copy(labels_hbm.at[pl.ds(base, CHUNK)], lab_v)
    z16 = jnp.zeros((16,), jnp.int32)
    cnt_v[pl.ds(0, 16)] = z16
    cnt_v[pl.ds(16, 16)] = z16
    lane = _lane()

    def step(i, _):
        v = lab_v[pl.ds(i * 16, 16)]
        occ, lastm = plsc.scan_count(v)
        basec = plsc.load_gather(cnt_v, [v])
        pos = basec + occ - 1
        keep = pos < M
        gidx = base + i * 16 + lane
        plsc.store_scatter(buf_v, [v, jnp.minimum(pos, M - 1)], gidx,
                           mask=keep)
        plsc.store_scatter(cnt_v, [v], basec + occ, mask=lastm)
        return 0

    lax.fori_loop(0, VREGS, step, 0)
    pltpu.sync_copy(cnt_v, loc_cnt_hbm.at[wid])
    copies = [pltpu.async_copy(buf_v.at[c], loc_idx_hbm.at[c, wid], sem)
              for c in range(NCLS)]
    for d in copies:
        d.wait()


def _k1(labels_flat):
    kern = pl.kernel(
        _k1_body,
        out_type=(jax.ShapeDtypeStruct((NW, NCP), jnp.int32),
                  jax.ShapeDtypeStruct((NCLS, NW, M), jnp.int32)),
        mesh=_mesh(),
        compiler_params=_cp,
        scratch_types=[pltpu.VMEM((CHUNK,), jnp.int32),
                       pltpu.VMEM((NCLS, M), jnp.int32),
                       pltpu.VMEM((NCP,), jnp.int32),
                       pltpu.SemaphoreType.DMA],
    )
    return kern(labels_flat)


def _k15_body(loc_cnt_hbm, loc_idx_hbm, glob_hbm, cnt_hbm,
              lc_v, cls_v, pre_v, glob_v, cw_v, sem):
    wid = _wid()
    lane = _lane()
    z16 = jnp.zeros((16,), jnp.int32)
    pltpu.sync_copy(loc_cnt_hbm, lc_v)

    @pl.when(wid == 0)
    def _():
        acc0 = z16
        acc1 = z16
        for w in range(NW):
            acc0 = acc0 + lc_v[w, pl.ds(0, 16)]
            acc1 = acc1 + lc_v[w, pl.ds(16, 16)]
        cw_v[pl.ds(0, 16)] = acc0
        cw_v[pl.ds(16, 16)] = acc1
        pltpu.sync_copy(cw_v, cnt_hbm)

    @pl.when(wid < NCLS)
    def _():
        c = wid
        pltpu.sync_copy(loc_idx_hbm.at[c], cls_v)
        fc = jnp.full((16,), c, jnp.int32)
        cw0 = plsc.load_gather(lc_v, [lane, fc])
        cw1 = plsc.load_gather(lc_v, [lane + 16, fc])
        t0 = jnp.sum(cw0)
        incl0 = plsc.cumsum(cw0)
        incl1 = plsc.cumsum(cw1) + t0
        total = t0 + jnp.sum(cw1)
        pre_v[pl.ds(0, 16)] = incl0 - cw0
        pre_v[pl.ds(16, 16)] = incl1 - cw1
        cap = jnp.minimum(total, M)

        e0 = incl0 - cw0
        e1 = incl1 - cw1

        def grp(g, _):
            j = g * 16 + lane
            acc = jnp.zeros((16,), jnp.int32)
            for w in range(16):
                acc = acc + (j >= e0[w]).astype(jnp.int32)
                acc = acc + (j >= e1[w]).astype(jnp.int32)
            wsel = acc - 1
            local = j - plsc.load_gather(pre_v, [wsel])
            local = jnp.clip(local, 0, M - 1)
            val = plsc.load_gather(cls_v, [wsel, local])
            val = jnp.where(j < cap, val, SENT)
            glob_v[pl.ds(g * 16, 16)] = val
            return 0

        lax.fori_loop(0, MPAD // 16, grp, 0)
        pltpu.sync_copy(glob_v, glob_hbm.at[c])


def _k15(loc_cnt, loc_idx):
    kern = pl.kernel(
        _k15_body,
        out_type=(jax.ShapeDtypeStruct((NCLS, MPAD), jnp.int32),
                  jax.ShapeDtypeStruct((NCP,), jnp.int32)),
        mesh=_mesh(),
        compiler_params=_cp,
        scratch_types=[pltpu.VMEM((NW, NCP), jnp.int32),
                       pltpu.VMEM((NW, M), jnp.int32),
                       pltpu.VMEM((NCP,), jnp.int32),
                       pltpu.VMEM((MPAD,), jnp.int32),
                       pltpu.VMEM((NCP,), jnp.int32),
                       pltpu.SemaphoreType.DMA],
    )
    return kern(loc_cnt, loc_idx)


def _k2_body(glob_hbm, cnt_hbm, dm_hbm, featsf_hbm, mt_hbm, res_hbm,
             glob_v, cnt_v, dm_v, mt_v,
             oidx_a, oidx_p, oidx_n, ocls_p, ocls_n,
             rows_a, rows_p, rows_n, res_v, sem):
    wid = _wid()
    lane = _lane()
    z16 = jnp.zeros((16,), jnp.int32)
    zf16 = jnp.zeros((16,), jnp.float32)

    @pl.when(wid >= NCLS - 1)
    def _():
        res_v[...] = zf16
        pltpu.sync_copy(res_v, res_hbm.at[wid])

    @pl.when(wid < NCLS - 1)
    def _():
        ii = wid + 1
        pltpu.sync_copy(glob_hbm, glob_v)
        pltpu.sync_copy(cnt_hbm, cnt_v)
        pltpu.sync_copy(dm_hbm, dm_v)
        pltpu.sync_copy(mt_hbm, mt_v)
        iiv = jnp.full((16,), ii, jnp.int32)
        c0 = lane
        c1 = lane + 16
        c1c = jnp.minimum(c1, NCLS - 1)
        cnt0 = cnt_v[pl.ds(0, 16)]
        cnt1 = cnt_v[pl.ds(16, 16)]
        dr0 = plsc.load_gather(dm_v, [iiv, c0])
        dr1 = plsc.load_gather(dm_v, [iiv, c1])
        adv0 = jnp.where(dr0 == 0.0, jnp.float32(256.0), dr0)
        adv1 = jnp.where(dr1 == 0.0, jnp.float32(256.0), dr1)
        exist0 = jnp.logical_and(cnt0 > 0, c0 != 0)
        exist1 = jnp.logical_and(cnt1 > 0, c1 < NCLS)
        inf = jnp.float32(jnp.inf)
        mn = jnp.minimum(jnp.min(jnp.where(exist0, adv0, inf)),
                         jnp.min(jnp.where(exist1, adv1, inf)))
        sel0 = jnp.logical_and(exist0, adv0 == mn)
        sel1 = jnp.logical_and(exist1, adv1 == mn)
        neg0 = (c0 != 0) & (c0 != ii) & jnp.logical_not(sel0)
        neg1 = (c1 < NCLS) & (c1 != ii) & jnp.logical_not(sel1)
        pos_tot = (jnp.sum(jnp.where(sel0, cnt0, 0))
                   + jnp.sum(jnp.where(sel1, cnt1, 0)))
        neg_tot = (jnp.sum(jnp.where(neg0, cnt0, 0))
                   + jnp.sum(jnp.where(neg1, cnt1, 0)))
        cnt_ii = jnp.sum(jnp.where(c0 == ii, cnt0, 0)) + \
            jnp.sum(jnp.where(c1 == ii, cnt1, 0))
        mt = jnp.min(mt_v[...])
        ms = jnp.minimum(jnp.minimum(cnt_ii, pos_tot),
                         jnp.minimum(neg_tot, mt))

        for j2 in range(2):
            for g in range(7):
                oidx_a[j2, pl.ds(g * 16, 16)] = z16
                oidx_p[j2, pl.ds(g * 16, 16)] = z16
                oidx_n[j2, pl.ds(g * 16, 16)] = z16
        for g in range(MPAD // 16):
            ocls_p[pl.ds(g * 16, 16)] = z16
            ocls_n[pl.ds(g * 16, 16)] = z16

        def afill(g, _):
            kvec = g * 16 + lane
            val = plsc.load_gather(glob_v, [iiv, jnp.minimum(kvec, MPAD - 1)])
            mc = jnp.minimum(val, N - 1)
            row = (kvec >= 112).astype(jnp.int32)
            col = kvec - row * 112
            plsc.store_scatter(oidx_a, [row, col], mc)
            return 0

        lax.fori_loop(0, GPAD // 16, afill, 0)

        lane0 = lane == 0
        hp0 = jnp.where(sel0, plsc.load_gather(glob_v, [c0, z16]), SENT)
        hp1 = jnp.where(sel1, plsc.load_gather(glob_v, [c1c, z16]), SENT)
        hn0 = jnp.where(neg0, plsc.load_gather(glob_v, [c0, z16]), SENT)
        hn1 = jnp.where(neg1, plsc.load_gather(glob_v, [c1c, z16]), SENT)

        def mstep(k, carry):
            hp0, hp1, pp0, pp1, hn0, hn1, pn0, pn1 = carry
            row = (k >= 112).astype(jnp.int32)
            rowv = jnp.full((16,), row, jnp.int32)
            colv = jnp.full((16,), k - row * 112, jnp.int32)
            kv = jnp.full((16,), k, jnp.int32)

            def half(h0, h1, p0, p1, memb0, memb1, oidx_ref, ocls_ref):
                m = jnp.minimum(jnp.min(h0), jnp.min(h1))
                e0 = h0 == m
                f0 = plsc.all_reduce_ffs(e0)
                f1 = plsc.all_reduce_ffs(h1 == m)
                cvec = jnp.where(jnp.any(e0), f0, f1 + 16)
                mc = jnp.full((16,), jnp.minimum(m, N - 1), jnp.int32)
                plsc.store_scatter(oidx_ref, [rowv, colv], mc, mask=lane0)
                plsc.store_scatter(ocls_ref, [kv], cvec, mask=lane0)
                a0 = jnp.logical_and(memb0, c0 == cvec)
                a1 = jnp.logical_and(memb1, c1 == cvec)
                p0n = p0 + a0.astype(jnp.int32)
                p1n = p1 + a1.astype(jnp.int32)
                g0 = plsc.load_gather(glob_v, [c0, jnp.minimum(p0n, MPAD - 1)])
                g1 = plsc.load_gather(glob_v,
                                      [c1c, jnp.minimum(p1n, MPAD - 1)])
                return (jnp.where(a0, g0, h0), jnp.where(a1, g1, h1),
                        p0n, p1n)

            hp0, hp1, pp0, pp1 = half(hp0, hp1, pp0, pp1, sel0, sel1,
                                      oidx_p, ocls_p)
            hn0, hn1, pn0, pn1 = half(hn0, hn1, pn0, pn1, neg0, neg1,
                                      oidx_n, ocls_n)
            return hp0, hp1, pp0, pp1, hn0, hn1, pn0, pn1

        lax.fori_loop(0, M, mstep,
                      (hp0, hp1, z16, z16, hn0, hn1, z16, z16))

        descs = []
        for (oix, rws) in ((oidx_a, rows_a), (oidx_p, rows_p),
                           (oidx_n, rows_n)):
            for j2 in range(2):
                descs.append(pltpu.async_copy(
                    featsf_hbm.at[oix.at[j2]], rws.at[j2], sem))
        for d in descs:
            d.wait()

        ms_v = jnp.full((16,), ms, jnp.int32)

        def grp(g, tl_sum):
            kvec = g * 16 + lane
            row = (kvec >= 112).astype(jnp.int32)
            col = kvec - row * 112
            accp = zf16
            accn = zf16
            for ch in range(32):
                fch = jnp.full((16,), ch, jnp.int32)
                va = plsc.load_gather(rows_a, [row, col, fch])
                vp = plsc.load_gather(rows_p, [row, col, fch])
                vn = plsc.load_gather(rows_n, [row, col, fch])
                accp = accp + va * vp
                accn = accn + va * vn
            ocp = ocls_p[pl.ds(g * 16, 16)]
            ocn = ocls_n[pl.ds(g * 16, 16)]
            dpos = plsc.load_gather(dm_v, [iiv, ocp])
            dneg = plsc.load_gather(dm_v, [iiv, ocn])
            d_ap = 1.0 - accp
            d_an = 1.0 - accn
            tl = jnp.maximum(
                d_ap - d_an + 0.1 + (dneg - dpos) * 0.125, 0.0)
            tl = jnp.where(kvec < ms_v, tl, 0.0)
            return tl_sum + jnp.sum(tl)

        tl_sum = lax.fori_loop(0, MPAD // 16, grp, jnp.float32(0.0))
        active = ms > 0
        msf_v = jnp.full((16,), jnp.maximum(ms, 1).astype(jnp.float32))
        tl_mean = jnp.where(jnp.full((16,), active),
                            jnp.full((16,), tl_sum) / msf_v, zf16)
        act_f = jnp.where(active, jnp.float32(1.0), jnp.float32(0.0))
        vals = (tl_mean * (lane == 0).astype(jnp.float32)
                + act_f * (lane == 1).astype(jnp.float32))
        res_v[...] = vals
        pltpu.sync_copy(res_v, res_hbm.at[wid])


def _k2(glob, cnt, dm_pad, featsf, mt_vec):
    kern = pl.kernel(
        _k2_body,
        out_type=jax.ShapeDtypeStruct((NW, 16), jnp.float32),
        mesh=_mesh(),
        compiler_params=_cp_untiled,
        scratch_types=[pltpu.VMEM((NCLS, MPAD), jnp.int32),
                       pltpu.VMEM((NCP,), jnp.int32),
                       pltpu.VMEM((NCP, NCP), jnp.float32),
                       pltpu.VMEM((16,), jnp.int32),
                       pltpu.VMEM((2, 112), jnp.int32),
                       pltpu.VMEM((2, 112), jnp.int32),
                       pltpu.VMEM((2, 112), jnp.int32),
                       pltpu.VMEM((MPAD,), jnp.int32),
                       pltpu.VMEM((MPAD,), jnp.int32),
                       pltpu.VMEM((2, 112, 32), jnp.float32),
                       pltpu.VMEM((2, 112, 32), jnp.float32),
                       pltpu.VMEM((2, 112, 32), jnp.float32),
                       pltpu.VMEM((16,), jnp.float32),
                       pltpu.SemaphoreType.DMA],
    )
    return kern(glob, cnt, dm_pad, featsf, mt_vec)


def _k3_body(res_hbm, out_hbm, res_v, out_v):
    wid = _wid()
    lane = _lane()

    @pl.when(wid == 0)
    def _():
        pltpu.sync_copy(res_hbm, res_v)
        z16 = jnp.zeros((16,), jnp.int32)
        o16 = jnp.full((16,), 1, jnp.int32)
        tl0 = plsc.load_gather(res_v, [lane, z16])
        tl1 = plsc.load_gather(res_v, [lane + 16, z16])
        ac0 = plsc.load_gather(res_v, [lane, o16])
        ac1 = plsc.load_gather(res_v, [lane + 16, o16])
        total = jnp.sum(tl0) + jnp.sum(tl1)
        count = jnp.sum(ac0) + jnp.sum(ac1)
        total_v = jnp.full((16,), total)
        count_v = jnp.full((16,), count)
        result_v = jnp.where(count_v == 0.0, jnp.float32(0.0),
                             total_v / jnp.maximum(count_v, 1.0))
        out_v[...] = result_v
        pltpu.sync_copy(out_v, out_hbm)


def _k3(res):
    kern = pl.kernel(
        _k3_body,
        out_type=jax.ShapeDtypeStruct((16,), jnp.float32),
        mesh=_mesh(),
        compiler_params=_cp,
        scratch_types=[pltpu.VMEM((NW, 16), jnp.float32),
                       pltpu.VMEM((16,), jnp.float32)],
    )
    return kern(res)


def kernel(feats, labels, dist_mat, max_triplet):
    lab = labels[:, ::2, ::2, ::2].astype(jnp.int32)
    labels_flat = lab.reshape(-1)
    featsf = jnp.transpose(feats, (0, 2, 3, 4, 1)).reshape(-1, feats.shape[1])
    dm_pad = jnp.zeros((NCP, NCP), jnp.float32).at[:NCLS, :NCLS].set(
        dist_mat.astype(jnp.float32))
    mt_vec = jnp.full((16,), max_triplet, jnp.int32)

    loc_cnt, loc_idx = _k1(labels_flat)
    glob, cnt = _k15(loc_cnt, loc_idx)
    res = _k2(glob, cnt, dm_pad, featsf, mt_vec)
    out = _k3(res)
    return out[0]


# speedup vs baseline: 87.8662x; 1.0000x over previous
import functools

import jax
import jax.numpy as jnp
from jax import lax
from jax.experimental import pallas as pl
from jax.experimental.pallas import tpu as pltpu
from jax.experimental.pallas import tpu_sc as plsc

NCLS = 28
NCP = 32
M = 200
MPAD = 208
GPAD = 224
NW = 32
N = 2 * 16 * 128 * 128
CHUNK = N // NW
VREGS = CHUNK // 16
SENT = 0x3FFFFFFF

_mesh = functools.partial(
    plsc.VectorSubcoreMesh, core_axis_name="c", subcore_axis_name="s")
_cp = pltpu.CompilerParams(needs_layout_passes=False)
_cp_untiled = pltpu.CompilerParams(needs_layout_passes=False,
                                   use_tc_tiling_on_sc=False)


def _wid():
    return lax.axis_index("s") * 2 + lax.axis_index("c")


def _lane():
    return lax.iota(jnp.int32, 16)


def _k1_body(labels_hbm, loc_cnt_hbm, loc_idx_hbm, lab_v, buf_v, cnt_v, sem):
    wid = _wid()
    base = wid * CHUNK
    pltpu.sync_copy(labels_hbm.at[pl.ds(base, CHUNK)], lab_v)
    z16 = jnp.zeros((16,), jnp.int32)
    cnt_v[pl.ds(0, 16)] = z16
    cnt_v[pl.ds(16, 16)] = z16
    lane = _lane()

    def step(i, _):
        v = lab_v[pl.ds(i * 16, 16)]
        occ, lastm = plsc.scan_count(v)
        basec = plsc.load_gather(cnt_v, [v])
        pos = basec + occ - 1
        keep = pos < M
        gidx = base + i * 16 + lane
        plsc.store_scatter(buf_v, [v, jnp.minimum(pos, M - 1)], gidx,
                           mask=keep)
        plsc.store_scatter(cnt_v, [v], basec + occ, mask=lastm)
        return 0

    lax.fori_loop(0, VREGS, step, 0)
    pltpu.sync_copy(cnt_v, loc_cnt_hbm.at[wid])
    copies = [pltpu.async_copy(buf_v.at[c], loc_idx_hbm.at[c, wid], sem)
              for c in range(NCLS)]
    for d in copies:
        d.wait()


def _k1(labels_flat):
    kern = pl.kernel(
        _k1_body,
        out_type=(jax.ShapeDtypeStruct((NW, NCP), jnp.int32),
                  jax.ShapeDtypeStruct((NCLS, NW, M), jnp.int32)),
        mesh=_mesh(),
        compiler_params=_cp,
        scratch_types=[pltpu.VMEM((CHUNK,), jnp.int32),
                       pltpu.VMEM((NCLS, M), jnp.int32),
                       pltpu.VMEM((NCP,), jnp.int32),
                       pltpu.SemaphoreType.DMA],
    )
    return kern(labels_flat)


def _k15_body(loc_cnt_hbm, loc_idx_hbm, glob_hbm, cnt_hbm,
              lc_v, cls_v, pre_v, glob_v, cw_v, sem):
    wid = _wid()
    lane = _lane()
    z16 = jnp.zeros((16,), jnp.int32)
    pltpu.sync_copy(loc_cnt_hbm, lc_v)

    @pl.when(wid == 0)
    def _():
        acc0 = z16
        acc1 = z16
        for w in range(NW):
            acc0 = acc0 + lc_v[w, pl.ds(0, 16)]
            acc1 = acc1 + lc_v[w, pl.ds(16, 16)]
        cw_v[pl.ds(0, 16)] = acc0
        cw_v[pl.ds(16, 16)] = acc1
        pltpu.sync_copy(cw_v, cnt_hbm)

    @pl.when(wid < NCLS)
    def _():
        c = wid
        pltpu.sync_copy(loc_idx_hbm.at[c], cls_v)
        fc = jnp.full((16,), c, jnp.int32)
        cw0 = plsc.load_gather(lc_v, [lane, fc])
        cw1 = plsc.load_gather(lc_v, [lane + 16, fc])
        t0 = jnp.sum(cw0)
        incl0 = plsc.cumsum(cw0)
        incl1 = plsc.cumsum(cw1) + t0
        total = t0 + jnp.sum(cw1)
        pre_v[pl.ds(0, 16)] = incl0 - cw0
        pre_v[pl.ds(16, 16)] = incl1 - cw1
        cap = jnp.minimum(total, M)

        e0 = incl0 - cw0
        e1 = incl1 - cw1

        def grp(g, _):
            j = g * 16 + lane
            acc = jnp.zeros((16,), jnp.int32)
            for w in range(16):
                acc = acc + (j >= e0[w]).astype(jnp.int32)
                acc = acc + (j >= e1[w]).astype(jnp.int32)
            wsel = acc - 1
            local = j - plsc.load_gather(pre_v, [wsel])
            local = jnp.clip(local, 0, M - 1)
            val = plsc.load_gather(cls_v, [wsel, local])
            val = jnp.where(j < cap, val, SENT)
            glob_v[pl.ds(g * 16, 16)] = val
            return 0

        lax.fori_loop(0, MPAD // 16, grp, 0)
        pltpu.sync_copy(glob_v, glob_hbm.at[c])


def _k15(loc_cnt, loc_idx):
    kern = pl.kernel(
        _k15_body,
        out_type=(jax.ShapeDtypeStruct((NCLS, MPAD), jnp.int32),
                  jax.ShapeDtypeStruct((NCP,), jnp.int32)),
        mesh=_mesh(),
        compiler_params=_cp,
        scratch_types=[pltpu.VMEM((NW, NCP), jnp.int32),
                       pltpu.VMEM((NW, M), jnp.int32),
                       pltpu.VMEM((NCP,), jnp.int32),
                       pltpu.VMEM((MPAD,), jnp.int32),
                       pltpu.VMEM((NCP,), jnp.int32),
                       pltpu.SemaphoreType.DMA],
    )
    return kern(loc_cnt, loc_idx)


def _k2_body(glob_hbm, cnt_hbm, dm_hbm, featsf_hbm, mt_hbm, res_hbm,
             glob_v, cnt_v, dm_v, mt_v,
             oidx_a, oidx_p, oidx_n, ocls_p, ocls_n,
             rows_a, rows_p, rows_n, res_v, sem):
    wid = _wid()
    lane = _lane()
    z16 = jnp.zeros((16,), jnp.int32)
    zf16 = jnp.zeros((16,), jnp.float32)

    @pl.when(wid >= NCLS - 1)
    def _():
        res_v[...] = zf16
        pltpu.sync_copy(res_v, res_hbm.at[wid])

    @pl.when(wid < NCLS - 1)
    def _():
        ii = wid + 1
        pltpu.sync_copy(glob_hbm, glob_v)
        pltpu.sync_copy(cnt_hbm, cnt_v)
        pltpu.sync_copy(dm_hbm, dm_v)
        pltpu.sync_copy(mt_hbm, mt_v)
        iiv = jnp.full((16,), ii, jnp.int32)
        c0 = lane
        c1 = lane + 16
        c1c = jnp.minimum(c1, NCLS - 1)
        cnt0 = cnt_v[pl.ds(0, 16)]
        cnt1 = cnt_v[pl.ds(16, 16)]
        dr0 = plsc.load_gather(dm_v, [iiv, c0])
        dr1 = plsc.load_gather(dm_v, [iiv, c1])
        adv0 = jnp.where(dr0 == 0.0, jnp.float32(256.0), dr0)
        adv1 = jnp.where(dr1 == 0.0, jnp.float32(256.0), dr1)
        exist0 = jnp.logical_and(cnt0 > 0, c0 != 0)
        exist1 = jnp.logical_and(cnt1 > 0, c1 < NCLS)
        inf = jnp.float32(jnp.inf)
        mn = jnp.minimum(jnp.min(jnp.where(exist0, adv0, inf)),
                         jnp.min(jnp.where(exist1, adv1, inf)))
        sel0 = jnp.logical_and(exist0, adv0 == mn)
        sel1 = jnp.logical_and(exist1, adv1 == mn)
        neg0 = (c0 != 0) & (c0 != ii) & jnp.logical_not(sel0)
        neg1 = (c1 < NCLS) & (c1 != ii) & jnp.logical_not(sel1)
        pos_tot = (jnp.sum(jnp.where(sel0, cnt0, 0))
                   + jnp.sum(jnp.where(sel1, cnt1, 0)))
        neg_tot = (jnp.sum(jnp.where(neg0, cnt0, 0))
                   + jnp.sum(jnp.where(neg1, cnt1, 0)))
        cnt_ii = jnp.sum(jnp.where(c0 == ii, cnt0, 0)) + \
            jnp.sum(jnp.where(c1 == ii, cnt1, 0))
        mt = jnp.min(mt_v[...])
        ms = jnp.minimum(jnp.minimum(cnt_ii, pos_tot),
                         jnp.minimum(neg_tot, mt))

        for j2 in range(2):
            for g in range(7):
                oidx_a[j2, pl.ds(g * 16, 16)] = z16
                oidx_p[j2, pl.ds(g * 16, 16)] = z16
                oidx_n[j2, pl.ds(g * 16, 16)] = z16
        for g in range(MPAD // 16):
            ocls_p[pl.ds(g * 16, 16)] = z16
            ocls_n[pl.ds(g * 16, 16)] = z16

        def afill(g, _):
            kvec = g * 16 + lane
            val = plsc.load_gather(glob_v, [iiv, jnp.minimum(kvec, MPAD - 1)])
            mc = jnp.minimum(val, N - 1)
            row = (kvec >= 112).astype(jnp.int32)
            col = kvec - row * 112
            plsc.store_scatter(oidx_a, [row, col], mc)
            return 0

        lax.fori_loop(0, GPAD // 16, afill, 0)

        lane0 = lane == 0
        hp0 = jnp.where(sel0, plsc.load_gather(glob_v, [c0, z16]), SENT)
        hp1 = jnp.where(sel1, plsc.load_gather(glob_v, [c1c, z16]), SENT)
        hn0 = jnp.where(neg0, plsc.load_gather(glob_v, [c0, z16]), SENT)
        hn1 = jnp.where(neg1, plsc.load_gather(glob_v, [c1c, z16]), SENT)

        def mstep(k, carry):
            hp0, hp1, pp0, pp1, hn0, hn1, pn0, pn1 = carry
            row = (k >= 112).astype(jnp.int32)
            rowv = jnp.full((16,), row, jnp.int32)
            colv = jnp.full((16,), k - row * 112, jnp.int32)
            kv = jnp.full((16,), k, jnp.int32)

            def half(h0, h1, p0, p1, memb0, memb1, oidx_ref, ocls_ref):
                m = jnp.minimum(jnp.min(h0), jnp.min(h1))
                e0 = h0 == m
                f0 = plsc.all_reduce_ffs(e0)
                f1 = plsc.all_reduce_ffs(h1 == m)
                cvec = jnp.where(jnp.any(e0), f0, f1 + 16)
                mc = jnp.full((16,), jnp.minimum(m, N - 1), jnp.int32)
                plsc.store_scatter(oidx_ref, [rowv, colv], mc, mask=lane0)
                plsc.store_scatter(ocls_ref, [kv], cvec, mask=lane0)
                a0 = jnp.logical_and(memb0, c0 == cvec)
                a1 = jnp.logical_and(memb1, c1 == cvec)
                p0n = p0 + a0.astype(jnp.int32)
                p1n = p1 + a1.astype(jnp.int32)
                g0 = plsc.load_gather(glob_v, [c0, jnp.minimum(p0n, MPAD - 1)])
                g1 = plsc.load_gather(glob_v,
                                      [c1c, jnp.minimum(p1n, MPAD - 1)])
                return (jnp.where(a0, g0, h0), jnp.where(a1, g1, h1),
                        p0n, p1n)

            hp0, hp1, pp0, pp1 = half(hp0, hp1, pp0, pp1, sel0, sel1,
                                      oidx_p, ocls_p)
            hn0, hn1, pn0, pn1 = half(hn0, hn1, pn0, pn1, neg0, neg1,
                                      oidx_n, ocls_n)
            return hp0, hp1, pp0, pp1, hn0, hn1, pn0, pn1

        lax.fori_loop(0, M, mstep,
                      (hp0, hp1, z16, z16, hn0, hn1, z16, z16))

        descs = []
        for (oix, rws) in ((oidx_a, rows_a), (oidx_p, rows_p),
                           (oidx_n, rows_n)):
            for j2 in range(2):
                descs.append(pltpu.async_copy(
                    featsf_hbm.at[oix.at[j2]], rws.at[j2], sem))
        for d in descs:
            d.wait()

        ms_v = jnp.full((16,), ms, jnp.int32)

        def grp(g, tl_sum):
            kvec = g * 16 + lane
            row = (kvec >= 112).astype(jnp.int32)
            col = kvec - row * 112
            accp = zf16
            accn = zf16
            for ch in range(32):
                fch = jnp.full((16,), ch, jnp.int32)
                va = plsc.load_gather(rows_a, [row, col, fch])
                vp = plsc.load_gather(rows_p, [row, col, fch])
                vn = plsc.load_gather(rows_n, [row, col, fch])
                accp = accp + va * vp
                accn = accn + va * vn
            ocp = ocls_p[pl.ds(g * 16, 16)]
            ocn = ocls_n[pl.ds(g * 16, 16)]
            dpos = plsc.load_gather(dm_v, [iiv, ocp])
            dneg = plsc.load_gather(dm_v, [iiv, ocn])
            d_ap = 1.0 - accp
            d_an = 1.0 - accn
            tl = jnp.maximum(
                d_ap - d_an + 0.1 + (dneg - dpos) * 0.125, 0.0)
            tl = jnp.where((kvec < ms_v) & (kvec < M), tl, 0.0)
            return tl_sum + jnp.sum(tl)

        tl_sum = lax.fori_loop(0, MPAD // 16, grp, jnp.float32(0.0))
        active = ms > 0
        msf_v = jnp.full((16,), jnp.maximum(ms, 1).astype(jnp.float32))
        tl_mean = jnp.where(jnp.full((16,), active),
                            jnp.full((16,), tl_sum) / msf_v, zf16)
        act_f = jnp.where(active, jnp.float32(1.0), jnp.float32(0.0))
        vals = (tl_mean * (lane == 0).astype(jnp.float32)
                + act_f * (lane == 1).astype(jnp.float32))
        res_v[...] = vals
        pltpu.sync_copy(res_v, res_hbm.at[wid])


def _k2(glob, cnt, dm_pad, featsf, mt_vec):
    kern = pl.kernel(
        _k2_body,
        out_type=jax.ShapeDtypeStruct((NW, 16), jnp.float32),
        mesh=_mesh(),
        compiler_params=_cp_untiled,
        scratch_types=[pltpu.VMEM((NCLS, MPAD), jnp.int32),
                       pltpu.VMEM((NCP,), jnp.int32),
                       pltpu.VMEM((NCP, NCP), jnp.float32),
                       pltpu.VMEM((16,), jnp.int32),
                       pltpu.VMEM((2, 112), jnp.int32),
                       pltpu.VMEM((2, 112), jnp.int32),
                       pltpu.VMEM((2, 112), jnp.int32),
                       pltpu.VMEM((MPAD,), jnp.int32),
                       pltpu.VMEM((MPAD,), jnp.int32),
                       pltpu.VMEM((2, 112, 32), jnp.float32),
                       pltpu.VMEM((2, 112, 32), jnp.float32),
                       pltpu.VMEM((2, 112, 32), jnp.float32),
                       pltpu.VMEM((16,), jnp.float32),
                       pltpu.SemaphoreType.DMA],
    )
    return kern(glob, cnt, dm_pad, featsf, mt_vec)


def _k3_body(res_hbm, out_hbm, res_v, out_v):
    wid = _wid()
    lane = _lane()

    @pl.when(wid == 0)
    def _():
        pltpu.sync_copy(res_hbm, res_v)
        z16 = jnp.zeros((16,), jnp.int32)
        o16 = jnp.full((16,), 1, jnp.int32)
        tl0 = plsc.load_gather(res_v, [lane, z16])
        tl1 = plsc.load_gather(res_v, [lane + 16, z16])
        ac0 = plsc.load_gather(res_v, [lane, o16])
        ac1 = plsc.load_gather(res_v, [lane + 16, o16])
        total = jnp.sum(tl0) + jnp.sum(tl1)
        count = jnp.sum(ac0) + jnp.sum(ac1)
        total_v = jnp.full((16,), total)
        count_v = jnp.full((16,), count)
        result_v = jnp.where(count_v == 0.0, jnp.float32(0.0),
                             total_v / jnp.maximum(count_v, 1.0))
        out_v[...] = result_v
        pltpu.sync_copy(out_v, out_hbm)


def _k3(res):
    kern = pl.kernel(
        _k3_body,
        out_type=jax.ShapeDtypeStruct((16,), jnp.float32),
        mesh=_mesh(),
        compiler_params=_cp,
        scratch_types=[pltpu.VMEM((NW, 16), jnp.float32),
                       pltpu.VMEM((16,), jnp.float32)],
    )
    return kern(res)


def kernel(feats, labels, dist_mat, max_triplet):
    lab = labels[:, ::2, ::2, ::2].astype(jnp.int32)
    labels_flat = lab.reshape(-1)
    featsf = jnp.transpose(feats, (0, 2, 3, 4, 1)).reshape(-1, feats.shape[1])
    dm_pad = jnp.zeros((NCP, NCP), jnp.float32).at[:NCLS, :NCLS].set(
        dist_mat.astype(jnp.float32))
    mt_vec = jnp.full((16,), max_triplet, jnp.int32)

    loc_cnt, loc_idx = _k1(labels_flat)
    glob, cnt = _k15(loc_cnt, loc_idx)
    res = _k2(glob, cnt, dm_pad, featsf, mt_vec)
    out = _k3(res)
    return out[0]


# speedup vs baseline: 88.0260x; 1.0018x over previous
import functools

import jax
import jax.numpy as jnp
from jax import lax
from jax.experimental import pallas as pl
from jax.experimental.pallas import tpu as pltpu
from jax.experimental.pallas import tpu_sc as plsc

NCLS = 28
NCP = 32
M = 200
MPAD = 208
GPAD = 224
NW = 32
N = 2 * 16 * 128 * 128
CHUNK = N // NW
VREGS = CHUNK // 16
SENT = 0x3FFFFFFF

_mesh = functools.partial(
    plsc.VectorSubcoreMesh, core_axis_name="c", subcore_axis_name="s")
_cp = pltpu.CompilerParams(needs_layout_passes=False)
_cp_untiled = pltpu.CompilerParams(needs_layout_passes=False,
                                   use_tc_tiling_on_sc=False)


def _wid():
    return lax.axis_index("s") * 2 + lax.axis_index("c")


def _lane():
    return lax.iota(jnp.int32, 16)


def _k1_body(labels_hbm, loc_cnt_hbm, loc_idx_hbm, lab_v, buf_v, cnt_v, sem):
    wid = _wid()
    base = wid * CHUNK
    pltpu.sync_copy(labels_hbm.at[pl.ds(base, CHUNK)], lab_v)
    z16 = jnp.zeros((16,), jnp.int32)
    cnt_v[pl.ds(0, 16)] = z16
    cnt_v[pl.ds(16, 16)] = z16
    lane = _lane()

    def step(i, _):
        v = lab_v[pl.ds(i * 16, 16)]
        occ, lastm = plsc.scan_count(v)
        basec = plsc.load_gather(cnt_v, [v])
        pos = basec + occ - 1
        keep = pos < M
        gidx = base + i * 16 + lane
        plsc.store_scatter(buf_v, [v, jnp.minimum(pos, M - 1)], gidx,
                           mask=keep)
        plsc.store_scatter(cnt_v, [v], basec + occ, mask=lastm)
        return 0

    lax.fori_loop(0, VREGS, step, 0)
    pltpu.sync_copy(cnt_v, loc_cnt_hbm.at[wid])
    copies = [pltpu.async_copy(buf_v.at[c], loc_idx_hbm.at[c, wid], sem)
              for c in range(NCLS)]
    for d in copies:
        d.wait()


def _k1(labels_flat):
    kern = pl.kernel(
        _k1_body,
        out_type=(jax.ShapeDtypeStruct((NW, NCP), jnp.int32),
                  jax.ShapeDtypeStruct((NCLS, NW, M), jnp.int32)),
        mesh=_mesh(),
        compiler_params=_cp,
        scratch_types=[pltpu.VMEM((CHUNK,), jnp.int32),
                       pltpu.VMEM((NCLS, M), jnp.int32),
                       pltpu.VMEM((NCP,), jnp.int32),
                       pltpu.SemaphoreType.DMA],
    )
    return kern(labels_flat)


def _k15_body(loc_cnt_hbm, loc_idx_hbm, glob_hbm, cnt_hbm,
              lc_v, cls_v, pre_v, glob_v, cw_v, sem):
    wid = _wid()
    lane = _lane()
    z16 = jnp.zeros((16,), jnp.int32)
    pltpu.sync_copy(loc_cnt_hbm, lc_v)

    @pl.when(wid == 0)
    def _():
        acc0 = z16
        acc1 = z16
        for w in range(NW):
            acc0 = acc0 + lc_v[w, pl.ds(0, 16)]
            acc1 = acc1 + lc_v[w, pl.ds(16, 16)]
        cw_v[pl.ds(0, 16)] = acc0
        cw_v[pl.ds(16, 16)] = acc1
        pltpu.sync_copy(cw_v, cnt_hbm)

    @pl.when(wid < NCLS)
    def _():
        c = wid
        pltpu.sync_copy(loc_idx_hbm.at[c], cls_v)
        fc = jnp.full((16,), c, jnp.int32)
        cw0 = plsc.load_gather(lc_v, [lane, fc])
        cw1 = plsc.load_gather(lc_v, [lane + 16, fc])
        t0 = jnp.sum(cw0)
        incl0 = plsc.cumsum(cw0)
        incl1 = plsc.cumsum(cw1) + t0
        total = t0 + jnp.sum(cw1)
        pre_v[pl.ds(0, 16)] = incl0 - cw0
        pre_v[pl.ds(16, 16)] = incl1 - cw1
        cap = jnp.minimum(total, M)

        e0 = incl0 - cw0
        e1 = incl1 - cw1

        def grp(g, _):
            j = g * 16 + lane
            acc = jnp.zeros((16,), jnp.int32)
            for w in range(16):
                acc = acc + (j >= e0[w]).astype(jnp.int32)
                acc = acc + (j >= e1[w]).astype(jnp.int32)
            wsel = acc - 1
            local = j - plsc.load_gather(pre_v, [wsel])
            local = jnp.clip(local, 0, M - 1)
            val = plsc.load_gather(cls_v, [wsel, local])
            val = jnp.where(j < cap, val, SENT)
            glob_v[pl.ds(g * 16, 16)] = val
            return 0

        lax.fori_loop(0, MPAD // 16, grp, 0)
        pltpu.sync_copy(glob_v, glob_hbm.at[c])


def _k15(loc_cnt, loc_idx):
    kern = pl.kernel(
        _k15_body,
        out_type=(jax.ShapeDtypeStruct((NCLS, MPAD), jnp.int32),
                  jax.ShapeDtypeStruct((NCP,), jnp.int32)),
        mesh=_mesh(),
        compiler_params=_cp,
        scratch_types=[pltpu.VMEM((NW, NCP), jnp.int32),
                       pltpu.VMEM((NW, M), jnp.int32),
                       pltpu.VMEM((NCP,), jnp.int32),
                       pltpu.VMEM((MPAD,), jnp.int32),
                       pltpu.VMEM((NCP,), jnp.int32),
                       pltpu.SemaphoreType.DMA],
    )
    return kern(loc_cnt, loc_idx)


def _k2_body(glob_hbm, cnt_hbm, dm_hbm, featsf_hbm, mt_hbm, res_hbm,
             glob_v, cnt_v, dm_v, mt_v,
             oidx_a, oidx_p, oidx_n, ocls_p, ocls_n,
             rows_a, rows_p, rows_n, res_v, sem):
    wid = _wid()
    lane = _lane()
    z16 = jnp.zeros((16,), jnp.int32)
    zf16 = jnp.zeros((16,), jnp.float32)

    @pl.when(wid >= NCLS - 1)
    def _():
        res_v[...] = zf16
        pltpu.sync_copy(res_v, res_hbm.at[wid])

    @pl.when(wid < NCLS - 1)
    def _():
        ii = wid + 1
        pltpu.sync_copy(glob_hbm, glob_v)
        pltpu.sync_copy(cnt_hbm, cnt_v)
        pltpu.sync_copy(dm_hbm, dm_v)
        pltpu.sync_copy(mt_hbm, mt_v)
        iiv = jnp.full((16,), ii, jnp.int32)
        c0 = lane
        c1 = lane + 16
        c1c = jnp.minimum(c1, NCLS - 1)
        cnt0 = cnt_v[pl.ds(0, 16)]
        cnt1 = cnt_v[pl.ds(16, 16)]
        dr0 = plsc.load_gather(dm_v, [iiv, c0])
        dr1 = plsc.load_gather(dm_v, [iiv, c1])
        adv0 = jnp.where(dr0 == 0.0, jnp.float32(256.0), dr0)
        adv1 = jnp.where(dr1 == 0.0, jnp.float32(256.0), dr1)
        exist0 = jnp.logical_and(cnt0 > 0, c0 != 0)
        exist1 = jnp.logical_and(cnt1 > 0, c1 < NCLS)
        inf = jnp.float32(jnp.inf)
        mn = jnp.minimum(jnp.min(jnp.where(exist0, adv0, inf)),
                         jnp.min(jnp.where(exist1, adv1, inf)))
        sel0 = jnp.logical_and(exist0, adv0 == mn)
        sel1 = jnp.logical_and(exist1, adv1 == mn)
        neg0 = (c0 != 0) & (c0 != ii) & jnp.logical_not(sel0)
        neg1 = (c1 < NCLS) & (c1 != ii) & jnp.logical_not(sel1)
        pos_tot = (jnp.sum(jnp.where(sel0, cnt0, 0))
                   + jnp.sum(jnp.where(sel1, cnt1, 0)))
        neg_tot = (jnp.sum(jnp.where(neg0, cnt0, 0))
                   + jnp.sum(jnp.where(neg1, cnt1, 0)))
        cnt_ii = jnp.sum(jnp.where(c0 == ii, cnt0, 0)) + \
            jnp.sum(jnp.where(c1 == ii, cnt1, 0))
        mt = jnp.min(mt_v[...])
        ms = jnp.minimum(jnp.minimum(cnt_ii, pos_tot),
                         jnp.minimum(neg_tot, mt))

        for j2 in range(2):
            for g in range(7):
                oidx_a[j2, pl.ds(g * 16, 16)] = z16
                oidx_p[j2, pl.ds(g * 16, 16)] = z16
                oidx_n[j2, pl.ds(g * 16, 16)] = z16
        for g in range(MPAD // 16):
            ocls_p[pl.ds(g * 16, 16)] = z16
            ocls_n[pl.ds(g * 16, 16)] = z16

        def afill(g, _):
            kvec = g * 16 + lane
            val = plsc.load_gather(glob_v, [iiv, jnp.minimum(kvec, MPAD - 1)])
            mc = jnp.minimum(val, N - 1)
            row = (kvec >= 112).astype(jnp.int32)
            col = kvec - row * 112
            plsc.store_scatter(oidx_a, [row, col], mc)
            return 0

        lax.fori_loop(0, GPAD // 16, afill, 0)

        descs = [pltpu.async_copy(featsf_hbm.at[oidx_a.at[j2]],
                                  rows_a.at[j2], sem) for j2 in range(2)]

        lane0 = lane == 0
        hp0 = jnp.where(sel0, plsc.load_gather(glob_v, [c0, z16]), SENT)
        hp1 = jnp.where(sel1, plsc.load_gather(glob_v, [c1c, z16]), SENT)
        hn0 = jnp.where(neg0, plsc.load_gather(glob_v, [c0, z16]), SENT)
        hn1 = jnp.where(neg1, plsc.load_gather(glob_v, [c1c, z16]), SENT)

        def mstep(k, carry):
            hp0, hp1, pp0, pp1, hn0, hn1, pn0, pn1 = carry
            row = (k >= 112).astype(jnp.int32)
            rowv = jnp.full((16,), row, jnp.int32)
            colv = jnp.full((16,), k - row * 112, jnp.int32)
            kv = jnp.full((16,), k, jnp.int32)

            def half(h0, h1, p0, p1, memb0, memb1, oidx_ref, ocls_ref):
                m = jnp.minimum(jnp.min(h0), jnp.min(h1))
                e0 = h0 == m
                f0 = plsc.all_reduce_ffs(e0)
                f1 = plsc.all_reduce_ffs(h1 == m)
                cvec = jnp.where(jnp.any(e0), f0, f1 + 16)
                mc = jnp.full((16,), jnp.minimum(m, N - 1), jnp.int32)
                plsc.store_scatter(oidx_ref, [rowv, colv], mc, mask=lane0)
                plsc.store_scatter(ocls_ref, [kv], cvec, mask=lane0)
                a0 = jnp.logical_and(memb0, c0 == cvec)
                a1 = jnp.logical_and(memb1, c1 == cvec)
                p0n = p0 + a0.astype(jnp.int32)
                p1n = p1 + a1.astype(jnp.int32)
                g0 = plsc.load_gather(glob_v, [c0, jnp.minimum(p0n, MPAD - 1)])
                g1 = plsc.load_gather(glob_v,
                                      [c1c, jnp.minimum(p1n, MPAD - 1)])
                return (jnp.where(a0, g0, h0), jnp.where(a1, g1, h1),
                        p0n, p1n)

            hp0, hp1, pp0, pp1 = half(hp0, hp1, pp0, pp1, sel0, sel1,
                                      oidx_p, ocls_p)
            hn0, hn1, pn0, pn1 = half(hn0, hn1, pn0, pn1, neg0, neg1,
                                      oidx_n, ocls_n)
            return hp0, hp1, pp0, pp1, hn0, hn1, pn0, pn1

        lax.fori_loop(0, M, mstep,
                      (hp0, hp1, z16, z16, hn0, hn1, z16, z16))

        for (oix, rws) in ((oidx_p, rows_p),
                           (oidx_n, rows_n)):
            for j2 in range(2):
                descs.append(pltpu.async_copy(
                    featsf_hbm.at[oix.at[j2]], rws.at[j2], sem))
        for d in descs:
            d.wait()

        ms_v = jnp.full((16,), ms, jnp.int32)

        def grp(g, tl_sum):
            kvec = g * 16 + lane
            row = (kvec >= 112).astype(jnp.int32)
            col = kvec - row * 112
            accp = zf16
            accn = zf16
            for ch in range(32):
                fch = jnp.full((16,), ch, jnp.int32)
                va = plsc.load_gather(rows_a, [row, col, fch])
                vp = plsc.load_gather(rows_p, [row, col, fch])
                vn = plsc.load_gather(rows_n, [row, col, fch])
                accp = accp + va * vp
                accn = accn + va * vn
            ocp = ocls_p[pl.ds(g * 16, 16)]
            ocn = ocls_n[pl.ds(g * 16, 16)]
            dpos = plsc.load_gather(dm_v, [iiv, ocp])
            dneg = plsc.load_gather(dm_v, [iiv, ocn])
            d_ap = 1.0 - accp
            d_an = 1.0 - accn
            tl = jnp.maximum(
                d_ap - d_an + 0.1 + (dneg - dpos) * 0.125, 0.0)
            tl = jnp.where((kvec < ms_v) & (kvec < M), tl, 0.0)
            return tl_sum + jnp.sum(tl)

        tl_sum = lax.fori_loop(0, MPAD // 16, grp, jnp.float32(0.0))
        active = ms > 0
        msf_v = jnp.full((16,), jnp.maximum(ms, 1).astype(jnp.float32))
        tl_mean = jnp.where(jnp.full((16,), active),
                            jnp.full((16,), tl_sum) / msf_v, zf16)
        act_f = jnp.where(active, jnp.float32(1.0), jnp.float32(0.0))
        vals = (tl_mean * (lane == 0).astype(jnp.float32)
                + act_f * (lane == 1).astype(jnp.float32))
        res_v[...] = vals
        pltpu.sync_copy(res_v, res_hbm.at[wid])


def _k2(glob, cnt, dm_pad, featsf, mt_vec):
    kern = pl.kernel(
        _k2_body,
        out_type=jax.ShapeDtypeStruct((NW, 16), jnp.float32),
        mesh=_mesh(),
        compiler_params=_cp_untiled,
        scratch_types=[pltpu.VMEM((NCLS, MPAD), jnp.int32),
                       pltpu.VMEM((NCP,), jnp.int32),
                       pltpu.VMEM((NCP, NCP), jnp.float32),
                       pltpu.VMEM((16,), jnp.int32),
                       pltpu.VMEM((2, 112), jnp.int32),
                       pltpu.VMEM((2, 112), jnp.int32),
                       pltpu.VMEM((2, 112), jnp.int32),
                       pltpu.VMEM((MPAD,), jnp.int32),
                       pltpu.VMEM((MPAD,), jnp.int32),
                       pltpu.VMEM((2, 112, 32), jnp.float32),
                       pltpu.VMEM((2, 112, 32), jnp.float32),
                       pltpu.VMEM((2, 112, 32), jnp.float32),
                       pltpu.VMEM((16,), jnp.float32),
                       pltpu.SemaphoreType.DMA],
    )
    return kern(glob, cnt, dm_pad, featsf, mt_vec)


def _k3_body(res_hbm, out_hbm, res_v, out_v):
    wid = _wid()
    lane = _lane()

    @pl.when(wid == 0)
    def _():
        pltpu.sync_copy(res_hbm, res_v)
        z16 = jnp.zeros((16,), jnp.int32)
        o16 = jnp.full((16,), 1, jnp.int32)
        tl0 = plsc.load_gather(res_v, [lane, z16])
        tl1 = plsc.load_gather(res_v, [lane + 16, z16])
        ac0 = plsc.load_gather(res_v, [lane, o16])
        ac1 = plsc.load_gather(res_v, [lane + 16, o16])
        total = jnp.sum(tl0) + jnp.sum(tl1)
        count = jnp.sum(ac0) + jnp.sum(ac1)
        total_v = jnp.full((16,), total)
        count_v = jnp.full((16,), count)
        result_v = jnp.where(count_v == 0.0, jnp.float32(0.0),
                             total_v / jnp.maximum(count_v, 1.0))
        out_v[...] = result_v
        pltpu.sync_copy(out_v, out_hbm)


def _k3(res):
    kern = pl.kernel(
        _k3_body,
        out_type=jax.ShapeDtypeStruct((16,), jnp.float32),
        mesh=_mesh(),
        compiler_params=_cp,
        scratch_types=[pltpu.VMEM((NW, 16), jnp.float32),
                       pltpu.VMEM((16,), jnp.float32)],
    )
    return kern(res)


def kernel(feats, labels, dist_mat, max_triplet):
    lab = labels[:, ::2, ::2, ::2].astype(jnp.int32)
    labels_flat = lab.reshape(-1)
    featsf = jnp.transpose(feats, (0, 2, 3, 4, 1)).reshape(-1, feats.shape[1])
    dm_pad = jnp.zeros((NCP, NCP), jnp.float32).at[:NCLS, :NCLS].set(
        dist_mat.astype(jnp.float32))
    mt_vec = jnp.full((16,), max_triplet, jnp.int32)

    loc_cnt, loc_idx = _k1(labels_flat)
    glob, cnt = _k15(loc_cnt, loc_idx)
    res = _k2(glob, cnt, dm_pad, featsf, mt_vec)
    out = _k3(res)
    return out[0]
